# baseline (speedup 1.0000x reference)
import jax
import jax.numpy as jnp
from jax import lax
from jax.experimental import pallas as pl

TOKENS = 32768
EMB_DIM = 768
NUM_EXPERTS = 64
BM = 4096


def _gate_kernel(h_ref, w_ref, b_ref, out_ref):
    g = lax.dot_general(
        h_ref[...], w_ref[...],
        dimension_numbers=(((1,), (1,)), ((), ())),
        preferred_element_type=jnp.float32,
    )
    g = g + b_ref[...]
    m1 = jnp.max(g, axis=1, keepdims=True)
    e = jnp.exp(g - m1)
    w = e / jnp.sum(e, axis=1, keepdims=True)
    col = lax.broadcasted_iota(jnp.int32, w.shape, 1)
    m1w = jnp.max(w, axis=1, keepdims=True)
    idx1 = jnp.min(jnp.where(w == m1w, col, NUM_EXPERTS), axis=1, keepdims=True)
    w2 = jnp.where(col == idx1, -jnp.inf, w)
    m2w = jnp.max(w2, axis=1, keepdims=True)
    idx2 = jnp.min(jnp.where(w2 == m2w, col, NUM_EXPERTS), axis=1, keepdims=True)
    wm = jnp.where((col == idx1) | (col == idx2), w, 0.0)
    denom = jnp.clip(jnp.sum(wm, axis=1, keepdims=True), 1e-9, None)
    out_ref[...] = wm / denom


@jax.jit
def kernel(h, W, b):
    b2 = b.reshape(1, NUM_EXPERTS)
    grid = (TOKENS // BM,)
    return pl.pallas_call(
        _gate_kernel,
        grid=grid,
        in_specs=[
            pl.BlockSpec((BM, EMB_DIM), lambda i: (i, 0)),
            pl.BlockSpec((NUM_EXPERTS, EMB_DIM), lambda i: (0, 0)),
            pl.BlockSpec((1, NUM_EXPERTS), lambda i: (0, 0)),
        ],
        out_specs=pl.BlockSpec((BM, NUM_EXPERTS), lambda i: (i, 0)),
        out_shape=jax.ShapeDtypeStruct((TOKENS, NUM_EXPERTS), jnp.float32),
    )(h, W, b2)


# speedup vs baseline: 1.2734x; 1.2734x over previous
import jax
import jax.numpy as jnp
from jax import lax
from jax.experimental import pallas as pl

TOKENS = 32768
EMB_DIM = 768
NUM_EXPERTS = 64
BM = 4096


def _gate_kernel(h_ref, w_ref, b_ref, out_ref):
    g = lax.dot_general(
        h_ref[...], w_ref[...],
        dimension_numbers=(((1,), (1,)), ((), ())),
        preferred_element_type=jnp.float32,
    )
    g = g + b_ref[...]
    m1 = jnp.max(g, axis=1, keepdims=True)
    m2 = jnp.max(jnp.where(g == m1, -jnp.inf, g), axis=1, keepdims=True)
    e = jnp.where(g >= m2, jnp.exp(g - m1), 0.0)
    out_ref[...] = e / jnp.sum(e, axis=1, keepdims=True)


@jax.jit
def kernel(h, W, b):
    b2 = b.reshape(1, NUM_EXPERTS)
    grid = (TOKENS // BM,)
    return pl.pallas_call(
        _gate_kernel,
        grid=grid,
        in_specs=[
            pl.BlockSpec((BM, EMB_DIM), lambda i: (i, 0)),
            pl.BlockSpec((NUM_EXPERTS, EMB_DIM), lambda i: (0, 0)),
            pl.BlockSpec((1, NUM_EXPERTS), lambda i: (0, 0)),
        ],
        out_specs=pl.BlockSpec((BM, NUM_EXPERTS), lambda i: (i, 0)),
        out_shape=jax.ShapeDtypeStruct((TOKENS, NUM_EXPERTS), jnp.float32),
    )(h, W, b2)


# speedup vs baseline: 1.2909x; 1.0137x over previous
import jax
import jax.numpy as jnp
from jax import lax
from jax.experimental import pallas as pl

TOKENS = 32768
EMB_DIM = 768
NUM_EXPERTS = 64
BM = 4096


def _gate_kernel(h_ref, w_ref, b_ref, out_ref):
    g = lax.dot_general(
        h_ref[...], w_ref[...],
        dimension_numbers=(((1,), (1,)), ((), ())),
        preferred_element_type=jnp.float32,
    )
    g = g + b_ref[...]
    m1 = jnp.max(g, axis=1, keepdims=True)
    m2 = jnp.max(jnp.where(g == m1, -jnp.inf, g), axis=1, keepdims=True)
    d = jnp.exp(m2 - m1)
    p1 = 1.0 / (1.0 + d)
    p2 = 1.0 - p1
    out_ref[...] = jnp.where(g == m1, p1, jnp.where(g == m2, p2, 0.0))


@jax.jit
def kernel(h, W, b):
    b2 = b.reshape(1, NUM_EXPERTS)
    grid = (TOKENS // BM,)
    return pl.pallas_call(
        _gate_kernel,
        grid=grid,
        in_specs=[
            pl.BlockSpec((BM, EMB_DIM), lambda i: (i, 0)),
            pl.BlockSpec((NUM_EXPERTS, EMB_DIM), lambda i: (0, 0)),
            pl.BlockSpec((1, NUM_EXPERTS), lambda i: (0, 0)),
        ],
        out_specs=pl.BlockSpec((BM, NUM_EXPERTS), lambda i: (i, 0)),
        out_shape=jax.ShapeDtypeStruct((TOKENS, NUM_EXPERTS), jnp.float32),
    )(h, W, b2)
